# Initial kernel scaffold; baseline (speedup 1.0000x reference)
import jax
import jax.numpy as jnp
from jax.experimental import pallas as pl


def kernel(features, table_0, table_1, table_2, table_3):
    raise NotImplementedError("write your pallas kernel here")


# speedup vs baseline: 7.7558x; 7.7558x over previous
import functools

import jax
import jax.numpy as jnp
from jax import lax
from jax.experimental import pallas as pl
from jax.experimental.pallas import tpu as pltpu
from jax.experimental.pallas import tpu_sc as plsc

T = 4
B = 4096
L = 50
D = 64
Z = 100000
NC = 2
NS = 16
NW = NC * NS
SPT = B // NW
CB = 16
CPT = SPT // CB
IDS = CB * L
KR = 8
KC = IDS // KR

_mesh = plsc.VectorSubcoreMesh(core_axis_name="c", subcore_axis_name="s")


@functools.partial(
    pl.kernel,
    out_type=jax.ShapeDtypeStruct((B, T * D), jnp.float32),
    mesh=_mesh,
    scratch_types=[
        pltpu.VMEM((KR, KC), jnp.int32),
        pltpu.VMEM((IDS, D), jnp.float32),
        pltpu.VMEM((CB, T * D), jnp.float32),
        pltpu.SemaphoreType.DMA,
    ],
    compiler_params=pltpu.CompilerParams(use_tc_tiling_on_sc=False),
)
def _emb(feat_hbm, t0, t1, t2, t3, out_hbm, fidx_v, rows_v, outb_v, sem):
    cid = lax.axis_index("c")
    sid = lax.axis_index("s")
    wid = sid * NC + cid
    tables = [t0, t1, t2, t3]

    def chunk_body(c, _):
        for f in range(T):
            tab = tables[f]
            g = wid * CPT + c
            pltpu.sync_copy(feat_hbm.at[f, g], fidx_v)

            def mod_body(k, _):
                for o in (0, 16, 32, 48, 64, 80, 84):
                    v = fidx_v[k, pl.ds(o, 16)]
                    fidx_v[k, pl.ds(o, 16)] = lax.rem(v, Z)
                return 0

            lax.fori_loop(0, KR, mod_body, 0)

            def gat_body(k, _, tab=tab):
                pltpu.make_async_copy(
                    tab.at[fidx_v.at[k]], rows_v.at[pl.ds(k * KC, KC)], sem
                ).start()
                return 0

            lax.fori_loop(0, KR, gat_body, 0)

            def wait_body(k, _, tab=tab):
                pltpu.make_async_copy(
                    tab.at[fidx_v.at[0]], rows_v.at[pl.ds(0, KC)], sem
                ).wait()
                return 0

            lax.fori_loop(0, KR, wait_body, 0)

            def bag_body(j, _, f=f):
                r0 = j * L

                def l_body(l, accs):
                    r = r0 + l
                    return tuple(
                        accs[d] + rows_v[r, pl.ds(d * 16, 16)] for d in range(4)
                    )

                accs = lax.fori_loop(
                    0, L, l_body,
                    tuple(jnp.zeros((16,), jnp.float32) for _ in range(4)),
                )
                for d in range(4):
                    outb_v[j, pl.ds(f * D + d * 16, 16)] = accs[d]
                return 0

            lax.fori_loop(0, CB, bag_body, 0)
        s0 = wid * SPT + c * CB
        pltpu.sync_copy(outb_v, out_hbm.at[pl.ds(s0, CB)])
        return 0

    lax.fori_loop(0, CPT, chunk_body, 0)


def kernel(features, table_0, table_1, table_2, table_3):
    feat4 = features.reshape(T, NW * CPT, KR, KC)
    return _emb(feat4, table_0, table_1, table_2, table_3)


# speedup vs baseline: 9.6278x; 1.2414x over previous
import functools

import jax
import jax.numpy as jnp
from jax import lax
from jax.experimental import pallas as pl
from jax.experimental.pallas import tpu as pltpu
from jax.experimental.pallas import tpu_sc as plsc

T = 4
B = 4096
L = 50
D = 64
Z = 100000
NC = 2
NS = 16
NW = NC * NS
SPT = B // NW
CB = 16
CPT = SPT // CB
IDS = CB * L
KR = 8
KC = IDS // KR

_mesh = plsc.VectorSubcoreMesh(core_axis_name="c", subcore_axis_name="s")


@functools.partial(
    pl.kernel,
    out_type=jax.ShapeDtypeStruct((B, T * D), jnp.float32),
    mesh=_mesh,
    scratch_types=[
        pltpu.VMEM((KR, KC), jnp.int32),
        pltpu.VMEM((KR, KC), jnp.int32),
        pltpu.VMEM((IDS, D), jnp.float32),
        pltpu.VMEM((IDS, D), jnp.float32),
        pltpu.VMEM((CB, T * D), jnp.float32),
        pltpu.SemaphoreType.DMA,
        pltpu.SemaphoreType.DMA,
    ],
    compiler_params=pltpu.CompilerParams(use_tc_tiling_on_sc=False),
)
def _emb(feat_hbm, t0, t1, t2, t3, out_hbm,
         fidx0, fidx1, rows0, rows1, outb_v, sem0, sem1):
    cid = lax.axis_index("c")
    sid = lax.axis_index("s")
    wid = sid * NC + cid
    tables = [t0, t1, t2, t3]
    fidx = [fidx0, fidx1]
    rows = [rows0, rows1]
    sems = [sem0, sem1]

    def stage(f, c, buf):
        tab = tables[f]
        fx = fidx[buf]
        rw = rows[buf]
        sm = sems[buf]
        g = wid * CPT + c
        pltpu.sync_copy(feat_hbm.at[f, g], fx)

        def mod_body(k, _):
            for o in (0, 16, 32, 48, 64, 80, 84):
                v = fx[k, pl.ds(o, 16)]
                fx[k, pl.ds(o, 16)] = lax.rem(v, Z)
            return 0

        lax.fori_loop(0, KR, mod_body, 0)

        def gat_body(k, _):
            pltpu.make_async_copy(
                tab.at[fx.at[k]], rw.at[pl.ds(k * KC, KC)], sm
            ).start()
            return 0

        lax.fori_loop(0, KR, gat_body, 0)

    def pool(f, c, buf):
        rw = rows[buf]
        pltpu.make_async_copy(
            tables[f].at[pl.ds(0, IDS)], rw, sems[buf]
        ).wait()

        def bag_body(j, _):
            r0 = j * L
            accs = tuple(rw[r0, pl.ds(d * 16, 16)] for d in range(4))

            def l_body(i, accs):
                base = r0 + 1 + i * 7
                for u in range(7):
                    r = base + u
                    accs = tuple(
                        accs[d] + rw[r, pl.ds(d * 16, 16)] for d in range(4)
                    )
                return accs

            accs = lax.fori_loop(0, 7, l_body, accs)
            for d in range(4):
                outb_v[j, pl.ds(f * D + d * 16, 16)] = accs[d]
            return 0

        lax.fori_loop(0, CB, bag_body, 0)

    def flush(c):
        s0 = wid * SPT + c * CB
        pltpu.sync_copy(outb_v, out_hbm.at[pl.ds(s0, CB)])

    stage(0, 0, 0)

    def chunk_body(c, _):
        for f in range(T):
            if f + 1 < T:
                stage(f + 1, c, (f + 1) % 2)
            else:
                stage(0, c + 1, 0)
            pool(f, c, f % 2)
        flush(c)
        return 0

    lax.fori_loop(0, CPT - 1, chunk_body, 0)

    c = CPT - 1
    for f in range(T):
        if f + 1 < T:
            stage(f + 1, c, (f + 1) % 2)
        pool(f, c, f % 2)
    flush(c)


def kernel(features, table_0, table_1, table_2, table_3):
    feat4 = features.reshape(T, NW * CPT, KR, KC)
    return _emb(feat4, table_0, table_1, table_2, table_3)


# speedup vs baseline: 11.8303x; 1.2288x over previous
import functools

import jax
import jax.numpy as jnp
from jax import lax
from jax.experimental import pallas as pl
from jax.experimental.pallas import tpu as pltpu
from jax.experimental.pallas import tpu_sc as plsc

T = 4
B = 4096
L = 50
D = 64
Z = 100000
NC = 2
NS = 16
NW = NC * NS
SPT = B // NW
CB = 16
CPT = SPT // CB
IDS = CB * L
KR = 8
KC = IDS // KR

_mesh = plsc.VectorSubcoreMesh(core_axis_name="c", subcore_axis_name="s")


@functools.partial(
    pl.kernel,
    out_type=jax.ShapeDtypeStruct((B, D), jnp.float32),
    mesh=_mesh,
    scratch_types=[
        pltpu.VMEM((KR, KC), jnp.int32),
        pltpu.VMEM((KR, KC), jnp.int32),
        pltpu.VMEM((IDS, D), jnp.float32),
        pltpu.VMEM((IDS, D), jnp.float32),
        pltpu.VMEM((CB, D), jnp.float32),
        pltpu.SemaphoreType.DMA,
        pltpu.SemaphoreType.DMA,
    ],
    compiler_params=pltpu.CompilerParams(use_tc_tiling_on_sc=False),
)
def _emb1(feat_hbm, tab, out_hbm, fidx0, fidx1, rows0, rows1, outb_v,
          sem0, sem1):
    cid = lax.axis_index("c")
    sid = lax.axis_index("s")
    wid = sid * NC + cid
    fidx = [fidx0, fidx1]
    rows = [rows0, rows1]
    sems = [sem0, sem1]

    def vmod(v):
        t = v & 31
        v5 = lax.shift_right_logical(v, 5)
        a = lax.shift_right_logical(v5, 13)
        b = v5 & 8191
        w = a * 1942 + b
        q = (w.astype(jnp.float32) * (1.0 / 3125.0)).astype(jnp.int32)
        r = w - q * 3125
        r = jnp.where(r < 0, r + 3125, r)
        r = jnp.where(r >= 3125, r - 3125, r)
        return lax.shift_left(lax.shift_left(r, 5) | t, 1)

    def stage(c, buf):
        fx = fidx[buf]
        rw = rows[buf]
        sm = sems[buf]
        pltpu.sync_copy(feat_hbm.at[wid * CPT + c], fx)

        def mod_body(k, _):
            for o in (0, 16, 32, 48, 64, 80):
                fx[k, pl.ds(o, 16)] = vmod(fx[k, pl.ds(o, 16)])
            v = fx[k, pl.ds(84, 16)]
            lane = lax.iota(jnp.int32, 16)
            fx[k, pl.ds(84, 16)] = jnp.where(lane >= 12, vmod(v), v)
            return 0

        lax.fori_loop(0, KR, mod_body, 0)

        def gat_body(k, _):
            pltpu.make_async_copy(
                tab.at[fx.at[k]], rw.at[pl.ds(k * KC, KC)], sm
            ).start()
            return 0

        lax.fori_loop(0, KR, gat_body, 0)

    def pool(c, buf):
        rw = rows[buf]
        pltpu.make_async_copy(tab.at[pl.ds(0, IDS)], rw, sems[buf]).wait()

        def bag_body(j, _):
            r0 = j * L
            accs = tuple(rw[r0, pl.ds(d * 16, 16)] for d in range(4))

            def l_body(i, accs):
                base = r0 + 1 + i * 7
                for u in range(7):
                    r = base + u
                    accs = tuple(
                        accs[d] + rw[r, pl.ds(d * 16, 16)] for d in range(4)
                    )
                return accs

            accs = lax.fori_loop(0, 7, l_body, accs)
            for d in range(4):
                outb_v[j, pl.ds(d * 16, 16)] = accs[d]
            return 0

        lax.fori_loop(0, CB, bag_body, 0)
        s0 = wid * SPT + c * CB
        pltpu.sync_copy(outb_v, out_hbm.at[pl.ds(s0, CB)])

    stage(0, 0)

    def pair_body(cc, _):
        c = cc * 2
        stage(c + 1, 1)
        pool(c, 0)
        stage(c + 2, 0)
        pool(c + 1, 1)
        return 0

    lax.fori_loop(0, CPT // 2 - 1, pair_body, 0)
    c = CPT - 2
    stage(c + 1, 1)
    pool(c, 0)
    pool(c + 1, 1)


def kernel(features, table_0, table_1, table_2, table_3):
    feats = features.reshape(T, NW * CPT, KR, KC)
    outs = []
    for f, t in enumerate((table_0, table_1, table_2, table_3)):
        tab_v = jnp.pad(t, ((0, 0), (0, D))).reshape(2 * Z, D)
        outs.append(_emb1(feats[f], tab_v))
    return jnp.concatenate(outs, axis=1)


# speedup vs baseline: 12.3320x; 1.0424x over previous
import functools

import jax
import jax.numpy as jnp
from jax import lax
from jax.experimental import pallas as pl
from jax.experimental.pallas import tpu as pltpu
from jax.experimental.pallas import tpu_sc as plsc

T = 4
B = 4096
L = 50
D = 64
Z = 100000
NC = 2
NS = 16
NW = NC * NS
SPT = B // NW
CB = 16
CPT = SPT // CB
IDS = CB * L
KR = 8
KC = IDS // KR

_mesh = plsc.VectorSubcoreMesh(core_axis_name="c", subcore_axis_name="s")


@functools.partial(
    pl.kernel,
    out_type=jax.ShapeDtypeStruct((B // 8, 8, 2, 128), jnp.float32),
    mesh=_mesh,
    scratch_types=[
        pltpu.VMEM((KR, KC), jnp.int32),
        pltpu.VMEM((KR, KC), jnp.int32),
        pltpu.VMEM((IDS, D), jnp.float32),
        pltpu.VMEM((IDS, D), jnp.float32),
        pltpu.VMEM((2, 8, 2, 128), jnp.float32),
        pltpu.SemaphoreType.DMA,
        pltpu.SemaphoreType.DMA,
    ],
    compiler_params=pltpu.CompilerParams(use_tc_tiling_on_sc=False),
)
def _emb(feat_hbm, t0, t1, t2, t3, out_hbm,
         fidx0, fidx1, rows0, rows1, outb_v, sem0, sem1):
    cid = lax.axis_index("c")
    sid = lax.axis_index("s")
    wid = sid * NC + cid
    tables = [t0, t1, t2, t3]
    fidx = [fidx0, fidx1]
    rows = [rows0, rows1]
    sems = [sem0, sem1]

    def vmod(v):
        t = v & 31
        v5 = lax.shift_right_logical(v, 5)
        a = lax.shift_right_logical(v5, 13)
        b = v5 & 8191
        w = a * 1942 + b
        q = (w.astype(jnp.float32) * (1.0 / 3125.0)).astype(jnp.int32)
        r = w - q * 3125
        r = jnp.where(r < 0, r + 3125, r)
        r = jnp.where(r >= 3125, r - 3125, r)
        return lax.shift_left(lax.shift_left(r, 5) | t, 1)

    def stage(f, c, buf):
        tab = tables[f]
        fx = fidx[buf]
        rw = rows[buf]
        sm = sems[buf]
        pltpu.sync_copy(feat_hbm.at[f, wid * CPT + c], fx)

        def mod_body(k, _):
            for o in (0, 16, 32, 48, 64, 80):
                fx[k, pl.ds(o, 16)] = vmod(fx[k, pl.ds(o, 16)])
            v = fx[k, pl.ds(84, 16)]
            lane = lax.iota(jnp.int32, 16)
            fx[k, pl.ds(84, 16)] = jnp.where(lane >= 12, vmod(v), v)
            return 0

        lax.fori_loop(0, KR, mod_body, 0)

        def gat_body(k, _):
            pltpu.make_async_copy(
                tab.at[fx.at[k]], rw.at[pl.ds(k * KC, KC)], sm
            ).start()
            return 0

        lax.fori_loop(0, KR, gat_body, 0)

    def pool(f, c, buf):
        rw = rows[buf]
        pltpu.make_async_copy(
            tables[f].at[pl.ds(0, IDS)], rw, sems[buf]
        ).wait()

        def bag_body(j, _):
            r0 = j * L
            accs = tuple(rw[r0, pl.ds(d * 16, 16)] for d in range(4))

            def l_body(i, accs):
                base = r0 + 1 + i * 7
                for u in range(7):
                    r = base + u
                    accs = tuple(
                        accs[d] + rw[r, pl.ds(d * 16, 16)] for d in range(4)
                    )
                return accs

            accs = lax.fori_loop(0, 7, l_body, accs)
            jb = lax.shift_right_logical(j, 3)
            jr = j & 7
            for d in range(4):
                col = f * D + d * 16
                outb_v[jb, jr, col // 128, pl.ds(col % 128, 16)] = accs[d]
            return 0

        lax.fori_loop(0, CB, bag_body, 0)

    def flush(c):
        rb0 = wid * (SPT // 8) + c * (CB // 8)
        pltpu.sync_copy(outb_v, out_hbm.at[pl.ds(rb0, CB // 8)])

    stage(0, 0, 0)

    def chunk_body(c, _):
        for f in range(T):
            if f + 1 < T:
                stage(f + 1, c, (f + 1) % 2)
            else:
                stage(0, c + 1, 0)
            pool(f, c, f % 2)
        flush(c)
        return 0

    lax.fori_loop(0, CPT - 1, chunk_body, 0)

    c = CPT - 1
    for f in range(T):
        if f + 1 < T:
            stage(f + 1, c, (f + 1) % 2)
        pool(f, c, f % 2)
    flush(c)


def kernel(features, table_0, table_1, table_2, table_3):
    feat4 = features.reshape(T, NW * CPT, KR, KC)
    tabs = [
        jnp.pad(t, ((0, 0), (0, D))).reshape(2 * Z, D)
        for t in (table_0, table_1, table_2, table_3)
    ]
    out4 = _emb(feat4, *tabs)
    return out4.reshape(B, T * D)
